# baseline (speedup 1.0000x reference)
import functools

import jax
import jax.numpy as jnp
from jax import lax
from jax.experimental import pallas as pl
from jax.experimental.pallas import tpu as pltpu
from jax.experimental.pallas import tpu_sc as plsc

OUT_H = 7
OUT_W = 7
SCALE = 0.25
GRID = 2
NSAMP = OUT_H * OUT_W * GRID * GRID * 4


def _index_body(H, W, rois_ref, idx_ref, w_ref):
    rois = rois_ref[...]
    batch = rois[:, 0:1].astype(jnp.int32)
    sx1 = rois[:, 1:2] * SCALE
    sy1 = rois[:, 2:3] * SCALE
    sx2 = rois[:, 3:4] * SCALE
    sy2 = rois[:, 4:5] * SCALE
    roi_w = jnp.maximum(sx2 - sx1, 1.0)
    roi_h = jnp.maximum(sy2 - sy1, 1.0)
    bin_h = roi_h / OUT_H
    bin_w = roi_w / OUT_W

    j = lax.broadcasted_iota(jnp.int32, (1, NSAMP), 1)
    bin_i = j // 16
    e = j % 16
    ph = (bin_i // OUT_W).astype(jnp.float32)
    pw = (bin_i % OUT_W).astype(jnp.float32)
    gy = e // 8
    gx = (e // 4) % 2
    a = (e // 2) % 2
    b = e % 2

    gyf = (gy.astype(jnp.float32) + 0.5) / GRID
    gxf = (gx.astype(jnp.float32) + 0.5) / GRID
    ys = sy1 + (ph + gyf) * bin_h
    xs = sx1 + (pw + gxf) * bin_w

    valid = (ys >= -1.0) & (ys <= H) & (xs >= -1.0) & (xs <= W)

    def axis_terms(v, sel_hi, dim):
        v = jnp.maximum(v, 0.0)
        l0 = jnp.floor(v).astype(jnp.int32)
        cond = l0 >= dim - 1
        low = jnp.where(cond, dim - 1, l0)
        high = jnp.where(cond, dim - 1, l0 + 1)
        v = jnp.where(cond, jnp.float32(dim - 1), v)
        lv = v - low.astype(jnp.float32)
        wt = jnp.where(sel_hi == 1, lv, 1.0 - lv)
        sel = jnp.where(sel_hi == 1, high, low)
        return sel, wt

    ysel, wy = axis_terms(ys, a, H)
    xsel, wx = axis_terms(xs, b, W)

    idx_ref[...] = batch * (H * W) + ysel * W + xsel
    w_ref[...] = wy * wx * valid.astype(jnp.float32) * 0.25


def _sc_body(C, RPT, feat_ref, idx_ref, w_ref, out_ref,
             idx_v, w_v, rows_v, outst_v, sem):
    NC = 2
    cid = lax.axis_index("c")
    sid = lax.axis_index("s")
    wid = sid * NC + cid
    chunk = RPT * NSAMP
    pltpu.sync_copy(idx_ref.at[pl.ds(wid * chunk, chunk)], idx_v)
    pltpu.sync_copy(w_ref.at[pl.ds(wid * chunk, chunk)], w_v)

    nvec = C // 16

    def roi_body(r, carry):
        def bin_body(bi, carry2):
            off = r * NSAMP + bi * 16
            idx_vec = idx_v[pl.ds(off, 16)]
            pltpu.async_copy(feat_ref.at[idx_vec], rows_v, sem).wait()
            w_vec = w_v[pl.ds(off, 16)]
            gdims = lax.GatherDimensionNumbers(
                offset_dims=(), collapsed_slice_dims=(0,), start_index_map=(0,))
            wb = [
                lax.gather(w_vec, jnp.full((16, 1), i, jnp.int32), gdims, (1,),
                           mode=lax.GatherScatterMode.PROMISE_IN_BOUNDS)
                for i in range(16)
            ]
            for cc in range(nvec):
                sl = pl.ds(cc * 16, 16)
                acc = wb[0] * rows_v[0, sl]
                for i in range(1, 16):
                    acc = acc + wb[i] * rows_v[i, sl]
                outst_v[pl.ds(bi * C + cc * 16, 16)] = acc
            return carry2

        lax.fori_loop(0, OUT_H * OUT_W, bin_body, 0, unroll=False)
        nbin = OUT_H * OUT_W
        pltpu.sync_copy(
            outst_v, out_ref.at[pl.ds((wid * RPT + r) * nbin * C, nbin * C)]
        )
        return carry

    lax.fori_loop(0, RPT, roi_body, 0, unroll=False)


def kernel(features, rois):
    N, C, H, W = features.shape
    R = rois.shape[0]
    NW = 32
    RP = ((R + NW - 1) // NW) * NW
    RPT = RP // NW
    nbin = OUT_H * OUT_W

    rois_p = jnp.pad(rois, ((0, RP - R), (0, 0)))
    feat = jnp.transpose(features, (0, 2, 3, 1)).reshape(N * H * W, C)

    idx, wts = pl.pallas_call(
        functools.partial(_index_body, H, W),
        out_shape=[
            jax.ShapeDtypeStruct((RP, NSAMP), jnp.int32),
            jax.ShapeDtypeStruct((RP, NSAMP), jnp.float32),
        ],
    )(rois_p)

    mesh = plsc.VectorSubcoreMesh(core_axis_name="c", subcore_axis_name="s")
    out_flat = pl.kernel(
        functools.partial(_sc_body, C, RPT),
        out_type=jax.ShapeDtypeStruct((RP * nbin * C,), jnp.float32),
        mesh=mesh,
        scratch_types=[
            pltpu.VMEM((RPT * NSAMP,), jnp.int32),
            pltpu.VMEM((RPT * NSAMP,), jnp.float32),
            pltpu.VMEM((16, C), jnp.float32),
            pltpu.VMEM((nbin * C,), jnp.float32),
            pltpu.SemaphoreType.DMA,
        ],
    )(feat, idx.reshape(-1), wts.reshape(-1))

    out = out_flat.reshape(RP, OUT_H, OUT_W, C)[:R]
    return jnp.transpose(out, (0, 3, 1, 2))


# speedup vs baseline: 2.3301x; 2.3301x over previous
import functools

import jax
import jax.numpy as jnp
from jax import lax
from jax.experimental import pallas as pl
from jax.experimental.pallas import tpu as pltpu
from jax.experimental.pallas import tpu_sc as plsc

OUT_H = 7
OUT_W = 7
SCALE = 0.25
GRID = 2
NSAMP = OUT_H * OUT_W * GRID * GRID * 4


def _index_body(H, W, rois_ref, idx_ref, w_ref):
    rois = rois_ref[...]
    batch = rois[:, 0:1].astype(jnp.int32)
    sx1 = rois[:, 1:2] * SCALE
    sy1 = rois[:, 2:3] * SCALE
    sx2 = rois[:, 3:4] * SCALE
    sy2 = rois[:, 4:5] * SCALE
    roi_w = jnp.maximum(sx2 - sx1, 1.0)
    roi_h = jnp.maximum(sy2 - sy1, 1.0)
    bin_h = roi_h / OUT_H
    bin_w = roi_w / OUT_W

    j = lax.broadcasted_iota(jnp.int32, (1, NSAMP), 1)
    bin_i = j // 16
    e = j % 16
    ph = (bin_i // OUT_W).astype(jnp.float32)
    pw = (bin_i % OUT_W).astype(jnp.float32)
    gy = e // 8
    gx = (e // 4) % 2
    a = (e // 2) % 2
    b = e % 2

    gyf = (gy.astype(jnp.float32) + 0.5) / GRID
    gxf = (gx.astype(jnp.float32) + 0.5) / GRID
    ys = sy1 + (ph + gyf) * bin_h
    xs = sx1 + (pw + gxf) * bin_w

    valid = (ys >= -1.0) & (ys <= H) & (xs >= -1.0) & (xs <= W)

    def axis_terms(v, sel_hi, dim):
        v = jnp.maximum(v, 0.0)
        l0 = jnp.floor(v).astype(jnp.int32)
        cond = l0 >= dim - 1
        low = jnp.where(cond, dim - 1, l0)
        high = jnp.where(cond, dim - 1, l0 + 1)
        v = jnp.where(cond, jnp.float32(dim - 1), v)
        lv = v - low.astype(jnp.float32)
        wt = jnp.where(sel_hi == 1, lv, 1.0 - lv)
        sel = jnp.where(sel_hi == 1, high, low)
        return sel, wt

    ysel, wy = axis_terms(ys, a, H)
    xsel, wx = axis_terms(xs, b, W)

    idx_ref[...] = batch * (H * W) + ysel * W + xsel
    w_ref[...] = wy * wx * valid.astype(jnp.float32) * 0.25


NBUF = 4


def _sc_body(C, RPT, feat_ref, idx_ref, w_ref, out_ref,
             idx_v, w_v, rows, stg, gsems, ssems):
    NC = 2
    wid = lax.axis_index("s") * NC + lax.axis_index("c")
    chunk = RPT * NSAMP
    pltpu.sync_copy(idx_ref.at[pl.ds(wid * chunk, chunk)], idx_v)
    pltpu.sync_copy(w_ref.at[pl.ds(wid * chunk, chunk)], w_v)

    total = RPT * OUT_H * OUT_W
    base = wid * total
    nvec = C // 16
    gdims = lax.GatherDimensionNumbers(
        offset_dims=(), collapsed_slice_dims=(0,), start_index_map=(0,))

    def issue_gather(t, rb, sem):
        idx_vec = idx_v[pl.ds(t * 16, 16)]
        pltpu.async_copy(feat_ref.at[idx_vec], rb, sem)

    for k in range(NBUF):
        issue_gather(k, rows[k], gsems[k])

    def compute(t, rb, sb):
        w_vec = w_v[pl.ds(t * 16, 16)]
        wb = [
            lax.gather(w_vec, jnp.full((16, 1), i, jnp.int32), gdims, (1,),
                       mode=lax.GatherScatterMode.PROMISE_IN_BOUNDS)
            for i in range(16)
        ]
        for cc in range(nvec):
            sl = pl.ds(cc * 16, 16)
            terms = [wb[i] * rb[i, sl] for i in range(16)]
            while len(terms) > 1:
                terms = [terms[i] + terms[i + 1]
                         for i in range(0, len(terms) - 1, 2)] + (
                             [terms[-1]] if len(terms) % 2 else [])
            sb[pl.ds(cc * 16, 16)] = terms[0]

    def body(u, carry):
        for k in range(NBUF):
            t = u * NBUF + k
            pltpu.make_async_copy(
                feat_ref.at[pl.ds(0, 16)], rows[k], gsems[k]).wait()

            @pl.when(u > 0)
            def _wait_store():
                pltpu.make_async_copy(
                    stg[k], out_ref.at[pl.ds(0, C)], ssems[k]).wait()

            compute(t, rows[k], stg[k])
            pltpu.async_copy(stg[k], out_ref.at[pl.ds((base + t) * C, C)],
                             ssems[k])

            @pl.when(t + NBUF < total)
            def _issue_next():
                issue_gather(t + NBUF, rows[k], gsems[k])

        return carry

    lax.fori_loop(0, total // NBUF, body, 0, unroll=False)
    for k in range(NBUF):
        pltpu.make_async_copy(stg[k], out_ref.at[pl.ds(0, C)], ssems[k]).wait()


def kernel(features, rois):
    N, C, H, W = features.shape
    R = rois.shape[0]
    NW = 32
    RP = ((R + NW - 1) // NW) * NW
    RPT = RP // NW
    nbin = OUT_H * OUT_W

    rois_p = jnp.pad(rois, ((0, RP - R), (0, 0)))
    feat = jnp.transpose(features, (0, 2, 3, 1)).reshape(N * H * W, C)

    idx, wts = pl.pallas_call(
        functools.partial(_index_body, H, W),
        out_shape=[
            jax.ShapeDtypeStruct((RP, NSAMP), jnp.int32),
            jax.ShapeDtypeStruct((RP, NSAMP), jnp.float32),
        ],
    )(rois_p)

    mesh = plsc.VectorSubcoreMesh(core_axis_name="c", subcore_axis_name="s")
    out_flat = pl.kernel(
        functools.partial(_sc_body, C, RPT),
        out_type=jax.ShapeDtypeStruct((RP * nbin * C,), jnp.float32),
        mesh=mesh,
        scratch_types=[
            pltpu.VMEM((RPT * NSAMP,), jnp.int32),
            pltpu.VMEM((RPT * NSAMP,), jnp.float32),
            [pltpu.VMEM((16, C), jnp.float32) for _ in range(NBUF)],
            [pltpu.VMEM((C,), jnp.float32) for _ in range(NBUF)],
            [pltpu.SemaphoreType.DMA for _ in range(NBUF)],
            [pltpu.SemaphoreType.DMA for _ in range(NBUF)],
        ],
    )(feat, idx.reshape(-1), wts.reshape(-1))

    out = out_flat.reshape(RP, OUT_H, OUT_W, C)[:R]
    return jnp.transpose(out, (0, 3, 1, 2))
